# baseline (speedup 1.0000x reference)
import functools

import jax
import jax.numpy as jnp
import numpy as np
from jax import lax
from jax.experimental import pallas as pl
from jax.experimental.pallas import tpu as pltpu
from jax.experimental.pallas import tpu_sc as plsc

N = 10000
E = 320000
D = 128
ED = 16
G = 64

NC = 2
NS = 16
NW = NC * NS
EPAD = 322560
EPW = EPAD // NW
C = 96
NCHUNK = EPW // C
NPAD = 10112
RPS = NPAD // NS


def _bn_body(x_ref, g_ref, b_ref, o_ref):
    x = x_ref[...]
    m = jnp.mean(x, axis=0, keepdims=True)
    v = jnp.mean((x - m) ** 2, axis=0, keepdims=True)
    o_ref[...] = (x - m) * jax.lax.rsqrt(v + 1e-5) * g_ref[...] + b_ref[...]


def _bn0(x, g, b):
    return pl.pallas_call(
        _bn_body,
        out_shape=jax.ShapeDtypeStruct((N, D), jnp.float32),
    )(x, g.reshape(1, D), b.reshape(1, D))


_EBLK = 4032


def _pack_bf16_words(u):
    outs = []
    for g in range(D // 32):
        a = u[:, 32 * g:32 * g + 16]
        b = u[:, 32 * g + 16:32 * g + 32]
        a16 = lax.bitcast_convert_type(a.astype(jnp.bfloat16), jnp.uint16)
        b16 = lax.bitcast_convert_type(b.astype(jnp.bfloat16), jnp.uint16)
        word = a16.astype(jnp.uint32) | (b16.astype(jnp.uint32) << 16)
        outs.append(lax.bitcast_convert_type(word, jnp.int32))
    return jnp.concatenate(outs, axis=1)


def _eproj_body(ea_ref, w_ref, b_ref, o_ref):
    ea = ea_ref[...]
    dn = (((1,), (1,)), ((), ()))
    o_ref[...] = _pack_bf16_words(
        lax.dot_general(ea, w_ref[...], dn,
                        preferred_element_type=jnp.float32) + b_ref[...])


def _eproj(ea, w, b):
    nblk = EPAD // _EBLK
    return pl.pallas_call(
        _eproj_body,
        grid=(nblk,),
        in_specs=[pl.BlockSpec((_EBLK, ED), lambda i: (i, 0)),
                  pl.BlockSpec((D, ED), lambda i: (0, 0)),
                  pl.BlockSpec((1, D), lambda i: (0, 0))],
        out_specs=pl.BlockSpec((_EBLK, D // 2), lambda i: (i, 0)),
        out_shape=jax.ShapeDtypeStruct((EPAD, D // 2), jnp.int32),
    )(ea, w, b.reshape(1, D))


def _mlp_body(final_relu, h_ref, p0_ref, p1_ref, w1_ref, b1_ref,
              g1_ref, bb1_ref, w2_ref, b2_ref, og_ref, ob_ref, o_ref):
    y = h_ref[...] + p0_ref[0] + p1_ref[0]
    dn = (((1,), (1,)), ((), ()))
    t = lax.dot_general(y, w1_ref[...], dn,
                        preferred_element_type=jnp.float32) + b1_ref[...]
    m = jnp.mean(t, axis=0, keepdims=True)
    v = jnp.mean((t - m) ** 2, axis=0, keepdims=True)
    t = (t - m) * lax.rsqrt(v + 1e-5) * g1_ref[...] + bb1_ref[...]
    t = jnp.maximum(t, 0.0)
    u = lax.dot_general(t, w2_ref[...], dn,
                        preferred_element_type=jnp.float32) + b2_ref[...]
    m2 = jnp.mean(u, axis=0, keepdims=True)
    v2 = jnp.mean((u - m2) ** 2, axis=0, keepdims=True)
    u = (u - m2) * lax.rsqrt(v2 + 1e-5) * og_ref[...] + ob_ref[...]
    if final_relu:
        u = jnp.maximum(u, 0.0)
    o_ref[...] = u


def _mlp(h, p, w1, b1, g1, bb1, w2, b2, og, ob, final_relu):
    nspec = pl.BlockSpec((N, D), lambda i: (0, 0))
    pspec = pl.BlockSpec((1, N, D), lambda i: (0, 0, 0))
    vspec = pl.BlockSpec((1, D), lambda i: (0, 0))
    wspec = pl.BlockSpec((D, D), lambda i: (0, 0))
    return pl.pallas_call(
        functools.partial(_mlp_body, final_relu),
        grid=(1,),
        in_specs=[nspec, pspec, pspec, wspec, vspec, vspec, vspec,
                  wspec, vspec, vspec, vspec],
        out_specs=nspec,
        out_shape=jax.ShapeDtypeStruct((N, D), jnp.float32),
    )(h, p[0:1], p[1:2], w1, b1.reshape(1, D), g1.reshape(1, D),
      bb1.reshape(1, D), w2, b2.reshape(1, D), og.reshape(1, D),
      ob.reshape(1, D))


def _pool_body(h_ref, bat_ref, o_ref):
    h = h_ref[...]
    bat = bat_ref[...]
    iota = lax.broadcasted_iota(jnp.int32, (N, G), 1)
    oh = (bat == iota).astype(jnp.float32)
    counts = jnp.sum(oh, axis=0, keepdims=True)
    ssum = lax.dot_general(oh, h, (((0,), (0,)), ((), ())),
                           preferred_element_type=jnp.float32)
    mean = ssum * (1.0 / jnp.maximum(counts, 1.0)).reshape(G, 1)
    rows = []
    for g in range(G):
        mg = jnp.max(jnp.where(bat == g, h, -jnp.inf), axis=0, keepdims=True)
        rows.append(mg)
    mx = jnp.concatenate(rows, axis=0)
    o_ref[:, :D] = mean
    o_ref[:, D:] = mx


def _pool(h, batch):
    return pl.pallas_call(
        _pool_body,
        out_shape=jax.ShapeDtypeStruct((G, 2 * D), jnp.float32),
    )(h, batch.reshape(N, 1))


def _sc_agg_body(x_hbm, src_hbm, dst_hbm, ep_hbm, out_hbm,
                 src0, dst0, ep0, xr0, sci0,
                 src1, dst1, ep1, xr1, sci1,
                 acc_sh, semf0, semg0, sems0, semf1, semg1, sems1):
    c = lax.axis_index("c")
    s = lax.axis_index("s")
    w = c * NS + s
    base = w * EPW
    KMAX = NCHUNK // 2 - 1

    A = (src0, dst0, ep0, xr0, sci0, semf0, semg0, sems0)
    B = (src1, dst1, ep1, xr1, sci1, semf1, semg1, sems1)

    def fetch(g, bufs):
        srcj, dstj, epj, _, _, semf, _, _ = bufs
        off = base + g * C
        pltpu.async_copy(src_hbm.at[pl.ds(off, C)], srcj, semf)
        pltpu.async_copy(dst_hbm.at[pl.ds(off, C)], dstj, semf)
        pltpu.async_copy(ep_hbm.at[pl.ds(off, C)], epj, semf)

    def fetch_wait(g, bufs):
        srcj, dstj, epj, _, _, semf, _, _ = bufs
        off = base + g * C
        pltpu.make_async_copy(src_hbm.at[pl.ds(off, C)], srcj, semf).wait()
        pltpu.make_async_copy(dst_hbm.at[pl.ds(off, C)], dstj, semf).wait()
        pltpu.make_async_copy(ep_hbm.at[pl.ds(off, C)], epj, semf).wait()

    def gather(bufs):
        srcj, _, _, xrj, _, _, semg, _ = bufs
        pltpu.async_copy(x_hbm.at[srcj], xrj, semg)

    def gather_wait(bufs):
        srcj, _, _, xrj, _, _, semg, _ = bufs
        pltpu.make_async_copy(x_hbm.at[srcj], xrj, semg).wait()

    def scat(bufs):
        _, dstj, _, xrj, scij, _, _, sems = bufs
        for j in range(C // 16):
            sl = pl.ds(j * 16, 16)
            scij[sl] = dstj[sl]
        pltpu.async_copy(xrj, acc_sh.at[scij], sems, add=True)

    def scat_wait(bufs):
        _, _, _, xrj, scij, _, _, sems = bufs
        pltpu.make_async_copy(xrj, acc_sh.at[scij], sems).wait()

    def compute(bufs):
        _, _, epj, xrj, _, _, _, _ = bufs

        def edge(i, carry):
            for j in range(D // 32):
                ev = plsc.bitcast(epj[i, pl.ds(j * 16, 16)], jnp.bfloat16)
                elo, ehi = plsc.unpack(ev,
                                       format=plsc.PackFormat.INTERLEAVED)
                lo = pl.ds(j * 32, 16)
                hi = pl.ds(j * 32 + 16, 16)
                xrj[i, lo] = jnp.maximum(xrj[i, lo] + elo, 0.0)
                xrj[i, hi] = jnp.maximum(xrj[i, hi] + ehi, 0.0)
            return carry

        lax.fori_loop(0, C, edge, 0, unroll=2)

    zero16 = jnp.zeros((16,), jnp.float32)

    def zrow(i, carry):
        for j in range(D // 16):
            xr0[i, pl.ds(j * 16, 16)] = zero16
        return carry

    lax.fori_loop(0, C, zrow, 0)
    for r in range(RPS // C):
        pltpu.sync_copy(xr0, acc_sh.at[pl.ds(s * RPS + r * C, C)])
    rem = RPS % C
    if rem:
        pltpu.sync_copy(xr0.at[pl.ds(0, rem)],
                        acc_sh.at[pl.ds(s * RPS + (RPS // C) * C, rem)])
    plsc.subcore_barrier()

    fetch(0, A)
    fetch_wait(0, A)
    gather(A)
    fetch(1, B)

    def pair(k, carry):
        g0 = 2 * k
        gather_wait(A)
        fetch_wait(g0 + 1, B)

        @pl.when(k >= 1)
        def _():
            scat_wait(B)

        gather(B)
        compute(A)
        scat(A)
        fetch(g0 + 2, A)
        gather_wait(B)
        fetch_wait(g0 + 2, A)
        scat_wait(A)
        gather(A)
        compute(B)
        scat(B)

        @pl.when(k < NCHUNK // 2 - 1)
        def _():
            fetch(g0 + 3, B)

        return carry

    lax.fori_loop(0, NCHUNK // 2, pair, 0)

    gather_wait(A)
    compute(A)
    scat(A)
    scat_wait(B)
    scat_wait(A)
    plsc.subcore_barrier()

    pltpu.sync_copy(acc_sh.at[pl.ds(s * RPS, RPS)],
                    out_hbm.at[c, pl.ds(s * RPS, RPS)])


def _sc_agg(x, src, dst, ep):
    mesh = plsc.VectorSubcoreMesh(core_axis_name="c", subcore_axis_name="s")
    ebufs = [
        pltpu.VMEM((C,), jnp.int32),
        pltpu.VMEM((C,), jnp.int32),
        pltpu.VMEM((C, D // 2), jnp.int32),
        pltpu.VMEM((C, D), jnp.float32),
        pltpu.VMEM((C,), jnp.int32),
    ]
    f = pl.kernel(
        _sc_agg_body,
        out_type=jax.ShapeDtypeStruct((NC, NPAD, D), jnp.float32),
        mesh=mesh,
        compiler_params=pltpu.CompilerParams(needs_layout_passes=False),
        scratch_types=ebufs + ebufs + [
            pltpu.VMEM_SHARED((NPAD, D), jnp.float32),
            pltpu.SemaphoreType.DMA,
            pltpu.SemaphoreType.DMA,
            pltpu.SemaphoreType.DMA,
            pltpu.SemaphoreType.DMA,
            pltpu.SemaphoreType.DMA,
            pltpu.SemaphoreType.DMA,
        ],
    )
    return f(x, src, dst, ep)


def kernel(x, edge_index, edge_attr, batch,
           bn0_g, bn0_b, bn1_g, bn1_b, bn2_g, bn2_b, bn3_g, bn3_b,
           e1_W, e1_b, m1_W1, m1_b1, m1_g, m1_bb, m1_W2, m1_b2,
           e2_W, e2_b, m2_W1, m2_b1, m2_g, m2_bb, m2_W2, m2_b2,
           e3_W, e3_b, m3_W1, m3_b1, m3_g, m3_bb, m3_W2, m3_b2):
    pad = EPAD - E
    ar = jnp.arange(pad, dtype=jnp.int32)
    src = jnp.concatenate([edge_index[0], ar % N])
    dst = jnp.concatenate([edge_index[1], N + ar % (NPAD - N)])
    ea = jnp.concatenate(
        [edge_attr, jnp.zeros((pad, ED), jnp.float32)], axis=0)

    h = _bn0(x, bn0_g, bn0_b)

    ep1 = _eproj(ea, e1_W, e1_b)
    p = _sc_agg(h, src, dst, ep1)
    ep2 = _eproj(ea, e2_W, e2_b)
    h = _mlp(h, p, m1_W1, m1_b1, m1_g, m1_bb, m1_W2, m1_b2,
             bn1_g, bn1_b, final_relu=True)

    p = _sc_agg(h, src, dst, ep2)
    ep3 = _eproj(ea, e3_W, e3_b)
    h = _mlp(h, p, m2_W1, m2_b1, m2_g, m2_bb, m2_W2, m2_b2,
             bn2_g, bn2_b, final_relu=True)

    p = _sc_agg(h, src, dst, ep3)
    h = _mlp(h, p, m3_W1, m3_b1, m3_g, m3_bb, m3_W2, m3_b2,
             bn3_g, bn3_b, final_relu=False)

    return _pool(h, batch)


# speedup vs baseline: 1.0005x; 1.0005x over previous
import functools

import jax
import jax.numpy as jnp
from jax import lax
from jax.experimental import pallas as pl
from jax.experimental.pallas import tpu as pltpu
from jax.experimental.pallas import tpu_sc as plsc

N = 10000
E = 320000
D = 128
ED = 16
G = 64

NC = 2
NS = 16
NW = NC * NS
EPAD = 322560
EPW = EPAD // NW
C = 96
NCHUNK = EPW // C
NPAD = 10112
RPS = NPAD // NS


def _bn_body(x_ref, g_ref, b_ref, o_ref):
    x = x_ref[...]
    m = jnp.mean(x, axis=0, keepdims=True)
    v = jnp.mean((x - m) ** 2, axis=0, keepdims=True)
    o_ref[...] = (x - m) * jax.lax.rsqrt(v + 1e-5) * g_ref[...] + b_ref[...]


def _bn0(x, g, b):
    return pl.pallas_call(
        _bn_body,
        out_shape=jax.ShapeDtypeStruct((N, D), jnp.float32),
    )(x, g.reshape(1, D), b.reshape(1, D))


_EBLK = 4032


def _pack_bf16_words(u):
    outs = []
    for g in range(D // 32):
        a = u[:, 32 * g:32 * g + 16]
        b = u[:, 32 * g + 16:32 * g + 32]
        a16 = lax.bitcast_convert_type(a.astype(jnp.bfloat16), jnp.uint16)
        b16 = lax.bitcast_convert_type(b.astype(jnp.bfloat16), jnp.uint16)
        word = a16.astype(jnp.uint32) | (b16.astype(jnp.uint32) << 16)
        outs.append(lax.bitcast_convert_type(word, jnp.int32))
    return jnp.concatenate(outs, axis=1)


def _eproj_body(ea_ref, w_ref, b_ref, o_ref):
    ea = ea_ref[...]
    dn = (((1,), (1,)), ((), ()))
    o_ref[...] = _pack_bf16_words(
        lax.dot_general(ea, w_ref[...], dn,
                        preferred_element_type=jnp.float32) + b_ref[...])


def _eproj(ea, w, b):
    nblk = EPAD // _EBLK
    return pl.pallas_call(
        _eproj_body,
        grid=(nblk,),
        in_specs=[pl.BlockSpec((_EBLK, ED), lambda i: (i, 0)),
                  pl.BlockSpec((D, ED), lambda i: (0, 0)),
                  pl.BlockSpec((1, D), lambda i: (0, 0))],
        out_specs=pl.BlockSpec((_EBLK, D // 2), lambda i: (i, 0)),
        out_shape=jax.ShapeDtypeStruct((EPAD, D // 2), jnp.int32),
    )(ea, w, b.reshape(1, D))


def _mlp_body(final_relu, h_ref, p0_ref, p1_ref, w1_ref, b1_ref,
              g1_ref, bb1_ref, w2_ref, b2_ref, og_ref, ob_ref, o_ref):
    y = h_ref[...] + p0_ref[0] + p1_ref[0]
    dn = (((1,), (1,)), ((), ()))
    t = lax.dot_general(y, w1_ref[...], dn,
                        preferred_element_type=jnp.float32) + b1_ref[...]
    m = jnp.mean(t, axis=0, keepdims=True)
    v = jnp.mean((t - m) ** 2, axis=0, keepdims=True)
    t = (t - m) * lax.rsqrt(v + 1e-5) * g1_ref[...] + bb1_ref[...]
    t = jnp.maximum(t, 0.0)
    u = lax.dot_general(t, w2_ref[...], dn,
                        preferred_element_type=jnp.float32) + b2_ref[...]
    m2 = jnp.mean(u, axis=0, keepdims=True)
    v2 = jnp.mean((u - m2) ** 2, axis=0, keepdims=True)
    u = (u - m2) * lax.rsqrt(v2 + 1e-5) * og_ref[...] + ob_ref[...]
    if final_relu:
        u = jnp.maximum(u, 0.0)
    o_ref[...] = u


def _mlp(h, p, w1, b1, g1, bb1, w2, b2, og, ob, final_relu):
    nspec = pl.BlockSpec((N, D), lambda i: (0, 0))
    pspec = pl.BlockSpec((1, N, D), lambda i: (0, 0, 0))
    vspec = pl.BlockSpec((1, D), lambda i: (0, 0))
    wspec = pl.BlockSpec((D, D), lambda i: (0, 0))
    return pl.pallas_call(
        functools.partial(_mlp_body, final_relu),
        grid=(1,),
        in_specs=[nspec, pspec, pspec, wspec, vspec, vspec, vspec,
                  wspec, vspec, vspec, vspec],
        out_specs=nspec,
        out_shape=jax.ShapeDtypeStruct((N, D), jnp.float32),
    )(h, p[0:1], p[1:2], w1, b1.reshape(1, D), g1.reshape(1, D),
      bb1.reshape(1, D), w2, b2.reshape(1, D), og.reshape(1, D),
      ob.reshape(1, D))


def _pool_body(h_ref, bat_ref, o_ref):
    h = h_ref[...]
    bat = bat_ref[...]
    iota = lax.broadcasted_iota(jnp.int32, (N, G), 1)
    oh = (bat == iota).astype(jnp.float32)
    counts = jnp.sum(oh, axis=0, keepdims=True)
    ssum = lax.dot_general(oh, h, (((0,), (0,)), ((), ())),
                           preferred_element_type=jnp.float32)
    mean = ssum * (1.0 / jnp.maximum(counts, 1.0)).reshape(G, 1)
    rows = []
    for g in range(G):
        mg = jnp.max(jnp.where(bat == g, h, -jnp.inf), axis=0, keepdims=True)
        rows.append(mg)
    mx = jnp.concatenate(rows, axis=0)
    o_ref[:, :D] = mean
    o_ref[:, D:] = mx


def _pool(h, batch):
    return pl.pallas_call(
        _pool_body,
        out_shape=jax.ShapeDtypeStruct((G, 2 * D), jnp.float32),
    )(h, batch.reshape(N, 1))


def _sc_agg_body(x_hbm, src_hbm, dst_hbm, ep_hbm, out_hbm,
                 src0, dst0, ep0, xr0, sci0,
                 src1, dst1, ep1, xr1, sci1,
                 acc_sh, semf0, semg0, sems0, semf1, semg1, sems1):
    c = lax.axis_index("c")
    s = lax.axis_index("s")
    w = c * NS + s
    base = w * EPW
    KMAX = NCHUNK // 2 - 1

    A = (src0, dst0, ep0, xr0, sci0, semf0, semg0, sems0)
    B = (src1, dst1, ep1, xr1, sci1, semf1, semg1, sems1)

    def fetch(g, bufs):
        srcj, dstj, epj, _, _, semf, _, _ = bufs
        off = base + g * C
        pltpu.async_copy(src_hbm.at[pl.ds(off, C)], srcj, semf)
        pltpu.async_copy(dst_hbm.at[pl.ds(off, C)], dstj, semf)
        pltpu.async_copy(ep_hbm.at[pl.ds(off, C)], epj, semf)

    def fetch_wait(g, bufs):
        srcj, dstj, epj, _, _, semf, _, _ = bufs
        off = base + g * C
        pltpu.make_async_copy(src_hbm.at[pl.ds(off, C)], srcj, semf).wait()
        pltpu.make_async_copy(dst_hbm.at[pl.ds(off, C)], dstj, semf).wait()
        pltpu.make_async_copy(ep_hbm.at[pl.ds(off, C)], epj, semf).wait()

    def gather(bufs):
        srcj, _, _, xrj, _, _, semg, _ = bufs
        pltpu.async_copy(x_hbm.at[srcj], xrj, semg)

    def gather_wait(bufs):
        srcj, _, _, xrj, _, _, semg, _ = bufs
        pltpu.make_async_copy(x_hbm.at[srcj], xrj, semg).wait()

    def scat(bufs):
        _, dstj, _, xrj, scij, _, _, sems = bufs
        for j in range(C // 16):
            sl = pl.ds(j * 16, 16)
            scij[sl] = dstj[sl]
        pltpu.async_copy(xrj, acc_sh.at[scij], sems, add=True)

    def scat_wait(bufs):
        _, _, _, xrj, scij, _, _, sems = bufs
        pltpu.make_async_copy(xrj, acc_sh.at[scij], sems).wait()

    def compute(bufs):
        _, _, epj, xrj, _, _, _, _ = bufs

        def edge(i, carry):
            for j in range(D // 32):
                ev = plsc.bitcast(epj[i, pl.ds(j * 16, 16)], jnp.bfloat16)
                elo, ehi = plsc.unpack(ev,
                                       format=plsc.PackFormat.INTERLEAVED)
                lo = pl.ds(j * 32, 16)
                hi = pl.ds(j * 32 + 16, 16)
                xrj[i, lo] = jnp.maximum(xrj[i, lo] + elo, 0.0)
                xrj[i, hi] = jnp.maximum(xrj[i, hi] + ehi, 0.0)
            return carry

        lax.fori_loop(0, C, edge, 0, unroll=2)

    zero16 = jnp.zeros((16,), jnp.float32)

    def zrow(i, carry):
        for j in range(D // 16):
            xr0[i, pl.ds(j * 16, 16)] = zero16
        return carry

    lax.fori_loop(0, C, zrow, 0)
    for r in range(RPS // C):
        pltpu.sync_copy(xr0, acc_sh.at[pl.ds(s * RPS + r * C, C)])
    rem = RPS % C
    if rem:
        pltpu.sync_copy(xr0.at[pl.ds(0, rem)],
                        acc_sh.at[pl.ds(s * RPS + (RPS // C) * C, rem)])
    plsc.subcore_barrier()

    fetch(0, A)
    fetch_wait(0, A)
    gather(A)
    fetch(1, B)

    def pair(k, carry):
        g0 = 2 * k
        gather_wait(A)
        fetch_wait(g0 + 1, B)

        @pl.when(k >= 1)
        def _():
            scat_wait(B)

        gather(B)
        compute(A)
        scat(A)
        fetch(g0 + 2, A)
        gather_wait(B)
        fetch_wait(g0 + 2, A)
        scat_wait(A)
        gather(A)
        compute(B)
        scat(B)

        @pl.when(k < NCHUNK // 2 - 1)
        def _():
            fetch(g0 + 3, B)

        return carry

    lax.fori_loop(0, NCHUNK // 2, pair, 0)

    gather_wait(A)
    compute(A)
    scat(A)
    scat_wait(B)
    scat_wait(A)
    plsc.subcore_barrier()

    pltpu.sync_copy(acc_sh.at[pl.ds(s * RPS, RPS)],
                    out_hbm.at[c, pl.ds(s * RPS, RPS)])


def _sc_agg(x, src, dst, ep):
    mesh = plsc.VectorSubcoreMesh(core_axis_name="c", subcore_axis_name="s")
    ebufs = [
        pltpu.VMEM((C,), jnp.int32),
        pltpu.VMEM((C,), jnp.int32),
        pltpu.VMEM((C, D // 2), jnp.int32),
        pltpu.VMEM((C, D), jnp.float32),
        pltpu.VMEM((C,), jnp.int32),
    ]
    f = pl.kernel(
        _sc_agg_body,
        out_type=jax.ShapeDtypeStruct((NC, NPAD, D), jnp.float32),
        mesh=mesh,
        compiler_params=pltpu.CompilerParams(needs_layout_passes=False),
        scratch_types=ebufs + ebufs + [
            pltpu.VMEM_SHARED((NPAD, D), jnp.float32),
            pltpu.SemaphoreType.DMA,
            pltpu.SemaphoreType.DMA,
            pltpu.SemaphoreType.DMA,
            pltpu.SemaphoreType.DMA,
            pltpu.SemaphoreType.DMA,
            pltpu.SemaphoreType.DMA,
        ],
    )
    return f(x, src, dst, ep)


def kernel(x, edge_index, edge_attr, batch,
           bn0_g, bn0_b, bn1_g, bn1_b, bn2_g, bn2_b, bn3_g, bn3_b,
           e1_W, e1_b, m1_W1, m1_b1, m1_g, m1_bb, m1_W2, m1_b2,
           e2_W, e2_b, m2_W1, m2_b1, m2_g, m2_bb, m2_W2, m2_b2,
           e3_W, e3_b, m3_W1, m3_b1, m3_g, m3_bb, m3_W2, m3_b2):
    pad = EPAD - E
    ar = jnp.arange(pad, dtype=jnp.int32)
    src = jnp.concatenate([edge_index[0], ar % N])
    dst = jnp.concatenate([edge_index[1], N + ar % (NPAD - N)])
    ea = jnp.concatenate(
        [edge_attr, jnp.zeros((pad, ED), jnp.float32)], axis=0)

    h = _bn0(x, bn0_g, bn0_b)

    ep1 = _eproj(ea, e1_W, e1_b)
    p = _sc_agg(h, src, dst, ep1)
    ep2 = _eproj(ea, e2_W, e2_b)
    h = _mlp(h, p, m1_W1, m1_b1, m1_g, m1_bb, m1_W2, m1_b2,
             bn1_g, bn1_b, final_relu=True)

    p = _sc_agg(h, src, dst, ep2)
    ep3 = _eproj(ea, e3_W, e3_b)
    h = _mlp(h, p, m2_W1, m2_b1, m2_g, m2_bb, m2_W2, m2_b2,
             bn2_g, bn2_b, final_relu=True)

    p = _sc_agg(h, src, dst, ep3)
    h = _mlp(h, p, m3_W1, m3_b1, m3_g, m3_bb, m3_W2, m3_b2,
             bn3_g, bn3_b, final_relu=False)

    return _pool(h, batch)
